# Initial kernel scaffold; baseline (speedup 1.0000x reference)
import jax
import jax.numpy as jnp
from jax.experimental import pallas as pl


def kernel(u, v, sim, mask_bool):
    raise NotImplementedError("write your pallas kernel here")


# speedup vs baseline: 1.0445x; 1.0445x over previous
import functools

import jax
import jax.numpy as jnp
from jax.experimental import pallas as pl
from jax.experimental.pallas import tpu as pltpu

_K = 16


def _smooth_body(idx_ref, val_ref, feat_ref, out_ref, *, tm, unroll=8):
    def chunk(it, carry):
        r0 = it * unroll
        accs = []
        for uu in range(unroll):
            r = r0 + uu
            acc = val_ref[r, 0] * feat_ref[idx_ref[r, 0]]
            for j in range(1, _K):
                acc = acc + val_ref[r, j] * feat_ref[idx_ref[r, j]]
            accs.append(acc)
        for uu in range(unroll):
            out_ref[r0 + uu] = accs[uu]
        return carry

    jax.lax.fori_loop(0, tm // unroll, chunk, 0)


def _smooth(idx, values, feat3, *, tm=256):
    n, _, d = feat3.shape
    tm = min(tm, n)
    grid = (n // tm,)
    return pl.pallas_call(
        functools.partial(_smooth_body, tm=tm),
        out_shape=jax.ShapeDtypeStruct((n, 1, d), jnp.float32),
        grid=grid,
        in_specs=[
            pl.BlockSpec((tm, _K), lambda i: (i, 0), memory_space=pltpu.SMEM),
            pl.BlockSpec((tm, _K), lambda i: (i, 0), memory_space=pltpu.SMEM),
            pl.BlockSpec((n, 1, d), lambda i: (0, 0, 0)),
        ],
        out_specs=pl.BlockSpec((tm, 1, d), lambda i: (i, 0, 0)),
        compiler_params=pltpu.CompilerParams(
            dimension_semantics=("parallel",),
            vmem_limit_bytes=48 * 1024 * 1024),
    )(idx, values, feat3)


def _decode_body(u_ref, v_ref, out_ref):
    x = jax.lax.dot_general(u_ref[...], v_ref[...],
                            dimension_numbers=(((1,), (1,)), ((), ())),
                            preferred_element_type=jnp.float32)
    out_ref[...] = jax.nn.sigmoid(x)


def _decode(u, v, *, tm=256, tn=512):
    su, d = u.shape
    sv, _ = v.shape
    tm = min(tm, su)
    tn = min(tn, sv)
    grid = (su // tm, sv // tn)
    return pl.pallas_call(
        _decode_body,
        out_shape=jax.ShapeDtypeStruct((su, sv), jnp.float32),
        grid=grid,
        in_specs=[
            pl.BlockSpec((tm, d), lambda i, j: (i, 0)),
            pl.BlockSpec((tn, d), lambda i, j: (j, 0)),
        ],
        out_specs=pl.BlockSpec((tm, tn), lambda i, j: (i, j)),
        compiler_params=pltpu.CompilerParams(
            dimension_semantics=("parallel", "parallel"),
            vmem_limit_bytes=48 * 1024 * 1024),
    )(u, v)


def kernel(u, v, sim, mask_bool):
    size_u, d = u.shape
    feature = jnp.concatenate([u, v], axis=0).astype(jnp.float32)
    n = feature.shape[0]

    values, idx = jax.lax.top_k(sim, _K)
    values = values.astype(jnp.float32)
    denom = values.sum(axis=1, keepdims=True)
    mask = mask_bool.reshape(n, 1)
    scaled = jnp.where(mask, 0.0, values / denom)
    col0 = jax.lax.broadcasted_iota(jnp.int32, (n, _K), 1) == 0
    scaled = jnp.where(mask & col0, 1.0, scaled)
    row_ids = jax.lax.broadcasted_iota(jnp.int32, (n, _K), 0)
    idx = jnp.where(mask, row_ids, idx.astype(jnp.int32))

    out3 = _smooth(idx, scaled, feature.reshape(n, 1, d))
    smoothed = out3.reshape(n, d)
    u_new = smoothed[:size_u]
    v_new = smoothed[size_u:]

    outputs = _decode(u_new.astype(jnp.bfloat16), v_new.astype(jnp.bfloat16))
    return outputs, u_new, v_new


# speedup vs baseline: 7.1574x; 6.8525x over previous
import functools

import jax
import jax.numpy as jnp
from jax.experimental import pallas as pl
from jax.experimental.pallas import tpu as pltpu

_K = 16


def _topk_body(sim_ref, mask_ref, idx_ref, val_ref, *, tm):
    x = sim_ref[...]
    col = jax.lax.broadcasted_iota(jnp.int32, x.shape, 1)
    vcols, icols = [], []
    for _ in range(_K):
        m = jnp.max(x, axis=1, keepdims=True)
        am = jnp.argmax(x, axis=1).astype(jnp.int32)[:, None]
        vcols.append(m)
        icols.append(am)
        x = jnp.where(col == am, -jnp.inf, x)
    vals = jnp.concatenate(vcols, axis=1)
    idx = jnp.concatenate(icols, axis=1)
    denom = jnp.sum(vals, axis=1, keepdims=True)
    mask = mask_ref[...] > 0.0
    scaled = jnp.where(mask, 0.0, vals / denom)
    kcol = jax.lax.broadcasted_iota(jnp.int32, vals.shape, 1)
    scaled = jnp.where(mask & (kcol == 0), 1.0, scaled)
    base = pl.program_id(0) * tm
    rows = base + jax.lax.broadcasted_iota(jnp.int32, idx.shape, 0)
    val_ref[...] = scaled
    idx_ref[...] = jnp.where(mask, rows, idx)


def _topk(sim, mask_f, *, tm=256):
    n, w = sim.shape
    tm = min(tm, n)
    grid = (n // tm,)
    return pl.pallas_call(
        functools.partial(_topk_body, tm=tm),
        out_shape=(jax.ShapeDtypeStruct((n, _K), jnp.int32),
                   jax.ShapeDtypeStruct((n, _K), jnp.float32)),
        grid=grid,
        in_specs=[
            pl.BlockSpec((tm, w), lambda i: (i, 0)),
            pl.BlockSpec((tm, 1), lambda i: (i, 0)),
        ],
        out_specs=(pl.BlockSpec((tm, _K), lambda i: (i, 0)),
                   pl.BlockSpec((tm, _K), lambda i: (i, 0))),
        compiler_params=pltpu.CompilerParams(
            dimension_semantics=("parallel",),
            vmem_limit_bytes=48 * 1024 * 1024),
    )(sim, mask_f)


def _smooth_body(idx_ref, val_ref, feat_ref, out_ref, *, tm, unroll=8):
    def chunk(it, carry):
        r0 = it * unroll
        accs = []
        for uu in range(unroll):
            r = r0 + uu
            acc = val_ref[r, 0] * feat_ref[idx_ref[r, 0]]
            for j in range(1, _K):
                acc = acc + val_ref[r, j] * feat_ref[idx_ref[r, j]]
            accs.append(acc)
        for uu in range(unroll):
            out_ref[r0 + uu] = accs[uu]
        return carry

    jax.lax.fori_loop(0, tm // unroll, chunk, 0)


def _smooth(idx, values, feat3, *, tm=256):
    n, _, d = feat3.shape
    tm = min(tm, n)
    grid = (n // tm,)
    return pl.pallas_call(
        functools.partial(_smooth_body, tm=tm),
        out_shape=jax.ShapeDtypeStruct((n, 1, d), jnp.float32),
        grid=grid,
        in_specs=[
            pl.BlockSpec((tm, _K), lambda i: (i, 0), memory_space=pltpu.SMEM),
            pl.BlockSpec((tm, _K), lambda i: (i, 0), memory_space=pltpu.SMEM),
            pl.BlockSpec((n, 1, d), lambda i: (0, 0, 0)),
        ],
        out_specs=pl.BlockSpec((tm, 1, d), lambda i: (i, 0, 0)),
        compiler_params=pltpu.CompilerParams(
            dimension_semantics=("parallel",),
            vmem_limit_bytes=48 * 1024 * 1024),
    )(idx, values, feat3)


def _decode_body(u_ref, v_ref, out_ref):
    x = jax.lax.dot_general(u_ref[...], v_ref[...],
                            dimension_numbers=(((1,), (1,)), ((), ())),
                            preferred_element_type=jnp.float32)
    out_ref[...] = jax.nn.sigmoid(x)


def _decode(u, v, *, tm=256, tn=512):
    su, d = u.shape
    sv, _ = v.shape
    tm = min(tm, su)
    tn = min(tn, sv)
    grid = (su // tm, sv // tn)
    return pl.pallas_call(
        _decode_body,
        out_shape=jax.ShapeDtypeStruct((su, sv), jnp.float32),
        grid=grid,
        in_specs=[
            pl.BlockSpec((tm, d), lambda i, j: (i, 0)),
            pl.BlockSpec((tn, d), lambda i, j: (j, 0)),
        ],
        out_specs=pl.BlockSpec((tm, tn), lambda i, j: (i, j)),
        compiler_params=pltpu.CompilerParams(
            dimension_semantics=("parallel", "parallel"),
            vmem_limit_bytes=48 * 1024 * 1024),
    )(u, v)


def kernel(u, v, sim, mask_bool):
    size_u, d = u.shape
    feature = jnp.concatenate([u, v], axis=0).astype(jnp.float32)
    n = feature.shape[0]

    mask_f = mask_bool.reshape(n, 1).astype(jnp.float32)
    idx, scaled = _topk(sim, mask_f)

    out3 = _smooth(idx, scaled, feature.reshape(n, 1, d))
    smoothed = out3.reshape(n, d)
    u_new = smoothed[:size_u]
    v_new = smoothed[size_u:]

    outputs = _decode(u_new.astype(jnp.bfloat16), v_new.astype(jnp.bfloat16))
    return outputs, u_new, v_new


# speedup vs baseline: 14.9488x; 2.0886x over previous
import functools

import jax
import jax.numpy as jnp
from jax.experimental import pallas as pl
from jax.experimental.pallas import tpu as pltpu

_K = 16


def _topk_body(sim_ref, mask_ref, idx_ref, val_ref, *, tm):
    x = sim_ref[...]
    ui = pltpu.bitcast(x, jnp.uint32)
    col = jax.lax.broadcasted_iota(jnp.uint32, x.shape, 1)
    enc = (ui & jnp.uint32(0xFFFFF000)) | (jnp.uint32(4095) - col)
    sim_ref[...] = pltpu.bitcast(enc, jnp.float32)
    vcols, icols = [], []
    for _ in range(_K):
        y = sim_ref[...]
        m = jnp.max(y, axis=1, keepdims=True)
        sim_ref[...] = jnp.where(y == m, -1.0, y)
        mui = pltpu.bitcast(m, jnp.uint32)
        icols.append((jnp.uint32(4095) - (mui & jnp.uint32(0xFFF)))
                     .astype(jnp.int32))
        vcols.append(pltpu.bitcast(mui & jnp.uint32(0xFFFFF000), jnp.float32))
    vals = jnp.concatenate(vcols, axis=1)
    idx = jnp.concatenate(icols, axis=1)
    denom = jnp.sum(vals, axis=1, keepdims=True)
    mask = mask_ref[...] > 0.0
    scaled = jnp.where(mask, 0.0, vals / denom)
    kcol = jax.lax.broadcasted_iota(jnp.int32, vals.shape, 1)
    scaled = jnp.where(mask & (kcol == 0), 1.0, scaled)
    base = pl.program_id(0) * tm
    rows = base + jax.lax.broadcasted_iota(jnp.int32, idx.shape, 0)
    val_ref[...] = scaled
    idx_ref[...] = jnp.where(mask, rows, idx)


def _topk(sim, mask_f, *, tm=512):
    n, w = sim.shape
    tm = min(tm, n)
    grid = (n // tm,)
    return pl.pallas_call(
        functools.partial(_topk_body, tm=tm),
        out_shape=(jax.ShapeDtypeStruct((n, _K), jnp.int32),
                   jax.ShapeDtypeStruct((n, _K), jnp.float32)),
        grid=grid,
        in_specs=[
            pl.BlockSpec((tm, w), lambda i: (i, 0)),
            pl.BlockSpec((tm, 1), lambda i: (i, 0)),
        ],
        out_specs=(pl.BlockSpec((tm, _K), lambda i: (i, 0)),
                   pl.BlockSpec((tm, _K), lambda i: (i, 0))),
        compiler_params=pltpu.CompilerParams(
            dimension_semantics=("parallel",),
            vmem_limit_bytes=48 * 1024 * 1024),
    )(sim, mask_f)


def _smooth_body(idx_ref, val_ref, feat_ref, out_ref, *, tm, unroll=8):
    def chunk(it, carry):
        r0 = it * unroll
        accs = []
        for uu in range(unroll):
            r = r0 + uu
            acc = val_ref[r, 0] * feat_ref[idx_ref[r, 0]]
            for j in range(1, _K):
                acc = acc + val_ref[r, j] * feat_ref[idx_ref[r, j]]
            accs.append(acc)
        for uu in range(unroll):
            out_ref[r0 + uu] = accs[uu]
        return carry

    jax.lax.fori_loop(0, tm // unroll, chunk, 0)


def _smooth(idx, values, feat3, *, tm=256):
    n, _, d = feat3.shape
    tm = min(tm, n)
    grid = (n // tm,)
    return pl.pallas_call(
        functools.partial(_smooth_body, tm=tm),
        out_shape=jax.ShapeDtypeStruct((n, 1, d), jnp.float32),
        grid=grid,
        in_specs=[
            pl.BlockSpec((tm, _K), lambda i: (i, 0), memory_space=pltpu.SMEM),
            pl.BlockSpec((tm, _K), lambda i: (i, 0), memory_space=pltpu.SMEM),
            pl.BlockSpec((n, 1, d), lambda i: (0, 0, 0)),
        ],
        out_specs=pl.BlockSpec((tm, 1, d), lambda i: (i, 0, 0)),
        compiler_params=pltpu.CompilerParams(
            dimension_semantics=("parallel",),
            vmem_limit_bytes=48 * 1024 * 1024),
    )(idx, values, feat3)


def _decode_body(u_ref, v_ref, out_ref):
    x = jax.lax.dot_general(u_ref[...], v_ref[...],
                            dimension_numbers=(((1,), (1,)), ((), ())),
                            preferred_element_type=jnp.float32)
    out_ref[...] = jax.nn.sigmoid(x)


def _decode(u, v, *, tm=256, tn=512):
    su, d = u.shape
    sv, _ = v.shape
    tm = min(tm, su)
    tn = min(tn, sv)
    grid = (su // tm, sv // tn)
    return pl.pallas_call(
        _decode_body,
        out_shape=jax.ShapeDtypeStruct((su, sv), jnp.float32),
        grid=grid,
        in_specs=[
            pl.BlockSpec((tm, d), lambda i, j: (i, 0)),
            pl.BlockSpec((tn, d), lambda i, j: (j, 0)),
        ],
        out_specs=pl.BlockSpec((tm, tn), lambda i, j: (i, j)),
        compiler_params=pltpu.CompilerParams(
            dimension_semantics=("parallel", "parallel"),
            vmem_limit_bytes=48 * 1024 * 1024),
    )(u, v)


def kernel(u, v, sim, mask_bool):
    size_u, d = u.shape
    feature = jnp.concatenate([u, v], axis=0).astype(jnp.float32)
    n = feature.shape[0]

    mask_f = mask_bool.reshape(n, 1).astype(jnp.float32)
    idx, scaled = _topk(sim, mask_f)

    out3 = _smooth(idx, scaled, feature.reshape(n, 1, d))
    smoothed = out3.reshape(n, d)
    u_new = smoothed[:size_u]
    v_new = smoothed[size_u:]

    outputs = _decode(u_new.astype(jnp.bfloat16), v_new.astype(jnp.bfloat16))
    return outputs, u_new, v_new
